# Initial kernel scaffold; baseline (speedup 1.0000x reference)
import jax
import jax.numpy as jnp
from jax.experimental import pallas as pl


def kernel(x_in, c1_w0, c1_b0, c1_w1, c1_b1, c1_w2, c1_b2, c2_w0, c2_b0, c2_w1, c2_b1, c2_w2, c2_b2, c3_w0, c3_b0, c3_w1, c3_b1, c3_w2, c3_b2):
    raise NotImplementedError("write your pallas kernel here")


# speedup vs baseline: 10.1571x; 10.1571x over previous
import jax
import jax.numpy as jnp
from jax.experimental import pallas as pl
from jax.experimental.pallas import tpu as pltpu

H, CH, W = 128, 64, 256
NE2 = 8
NE3 = 64
O1 = 8
O2 = 12
HID = 32


def _leaky(x):
    return jnp.where(x > 0, x, 0.01 * x)


def _argmax0(a, n):
    mx = jnp.max(a, axis=0)
    iota = jax.lax.broadcasted_iota(jnp.int32, a.shape, 0)
    cand = jnp.where(a == mx[None, :], iota, n)
    return jnp.min(cand, axis=0).astype(jnp.int32)


def _line_kernel(x_ref,
                 w10, b10, w11, b11, w12, b12,
                 w20, b20, w21, b21, w22, b22,
                 w30, b30, w31, b31, w32, b32,
                 out_ref):
    X = x_ref[0]

    h = _leaky(jnp.dot(w10[0], X, preferred_element_type=jnp.float32) + b10[0])
    h = _leaky(jnp.dot(w11[0], h, preferred_element_type=jnp.float32) + b11[0])
    s1 = jnp.dot(w12[0], h, preferred_element_type=jnp.float32) + b12[0]
    inds1 = _argmax0(s1, O1)

    e_iota2 = jax.lax.broadcasted_iota(jnp.int32, (NE2, 1, W), 0)
    m2 = e_iota2 == inds1[None, None, :]

    wt = jnp.swapaxes(w20[0], 1, 2).reshape(NE2 * HID, CH)
    g = jnp.dot(wt, X, preferred_element_type=jnp.float32)
    g = g.reshape(NE2, HID, W) + b20[0]
    h = _leaky(jnp.sum(jnp.where(m2, g, 0.0), axis=0))

    wt = jnp.swapaxes(w21[0], 1, 2).reshape(NE2 * HID, HID)
    g = jnp.dot(wt, h, preferred_element_type=jnp.float32)
    g = g.reshape(NE2, HID, W) + b21[0]
    h = _leaky(jnp.sum(jnp.where(m2, g, 0.0), axis=0))

    wt = jnp.swapaxes(w22[0], 1, 2).reshape(NE2 * O2, HID)
    g = jnp.dot(wt, h, preferred_element_type=jnp.float32)
    g = g.reshape(NE2, O2, W) + b22[0]
    s2 = jnp.sum(jnp.where(m2, g, 0.0), axis=0)

    inds2 = _argmax0(s2, O2)
    inds12_raw = inds1 * 8 + inds2 - 2
    inds12 = jnp.clip(inds12_raw, 0, NE3 - 1)

    e_iota3 = jax.lax.broadcasted_iota(jnp.int32, (NE3, 1, W), 0)
    m3 = e_iota3 == inds12[None, None, :]

    wt = jnp.swapaxes(w30[0], 1, 2).reshape(NE3 * HID, CH)
    g = jnp.dot(wt, X, preferred_element_type=jnp.float32)
    g = g.reshape(NE3, HID, W) + b30[0]
    h = _leaky(jnp.sum(jnp.where(m3, g, 0.0), axis=0))

    wt = jnp.swapaxes(w31[0], 1, 2).reshape(NE3 * HID, HID)
    g = jnp.dot(wt, h, preferred_element_type=jnp.float32)
    g = g.reshape(NE3, HID, W) + b31[0]
    h = _leaky(jnp.sum(jnp.where(m3, g, 0.0), axis=0))

    wt = jnp.swapaxes(w32[0], 1, 2).reshape(NE3 * O2, HID)
    g = jnp.dot(wt, h, preferred_element_type=jnp.float32)
    g = g.reshape(NE3, O2, W) + b32[0]
    s3 = jnp.sum(jnp.where(m3, g, 0.0), axis=0)

    inds3 = _argmax0(s3, O2)
    out_ref[0, 0] = jnp.clip(inds12_raw * 8 + inds3 - 2, 0, 511)


def kernel(x_in, c1_w0, c1_b0, c1_w1, c1_b1, c1_w2, c1_b2,
           c2_w0, c2_b0, c2_w1, c2_b1, c2_w2, c2_b2,
           c3_w0, c3_b0, c3_w1, c3_b1, c3_w2, c3_b2):
    x_t = jnp.transpose(x_in[0], (1, 0, 2))

    def wspec(e, i, o):
        return pl.BlockSpec((1, e, i, o), lambda h: (h, 0, 0, 0))

    def bspec(e, o):
        return pl.BlockSpec((1, e, o, 1), lambda h: (h, 0, 0, 0))

    in_specs = [
        pl.BlockSpec((1, CH, W), lambda h: (h, 0, 0)),
        pl.BlockSpec((1, HID, CH), lambda h: (h, 0, 0)),
        pl.BlockSpec((1, HID, 1), lambda h: (h, 0, 0)),
        pl.BlockSpec((1, HID, HID), lambda h: (h, 0, 0)),
        pl.BlockSpec((1, HID, 1), lambda h: (h, 0, 0)),
        pl.BlockSpec((1, O1, HID), lambda h: (h, 0, 0)),
        pl.BlockSpec((1, O1, 1), lambda h: (h, 0, 0)),
        wspec(NE2, CH, HID), bspec(NE2, HID),
        wspec(NE2, HID, HID), bspec(NE2, HID),
        wspec(NE2, HID, O2), bspec(NE2, O2),
        wspec(NE3, CH, HID), bspec(NE3, HID),
        wspec(NE3, HID, HID), bspec(NE3, HID),
        wspec(NE3, HID, O2), bspec(NE3, O2),
    ]

    args = [
        x_t,
        c1_w0, c1_b0.reshape(H, HID, 1),
        c1_w1, c1_b1.reshape(H, HID, 1),
        c1_w2, c1_b2.reshape(H, O1, 1),
        c2_w0.reshape(H, NE2, CH, HID), c2_b0.reshape(H, NE2, HID, 1),
        c2_w1.reshape(H, NE2, HID, HID), c2_b1.reshape(H, NE2, HID, 1),
        c2_w2.reshape(H, NE2, HID, O2), c2_b2.reshape(H, NE2, O2, 1),
        c3_w0.reshape(H, NE3, CH, HID), c3_b0.reshape(H, NE3, HID, 1),
        c3_w1.reshape(H, NE3, HID, HID), c3_b1.reshape(H, NE3, HID, 1),
        c3_w2.reshape(H, NE3, HID, O2), c3_b2.reshape(H, NE3, O2, 1),
    ]

    out = pl.pallas_call(
        _line_kernel,
        grid=(H,),
        in_specs=in_specs,
        out_specs=pl.BlockSpec((1, 1, W), lambda h: (h, 0, 0)),
        out_shape=jax.ShapeDtypeStruct((H, 1, W), jnp.int32),
        compiler_params=pltpu.CompilerParams(
            dimension_semantics=("arbitrary",),
        ),
    )(*args)

    return out.reshape(1, 1, H, W)
